# baseline (speedup 1.0000x reference)
import jax
import jax.numpy as jnp
from jax.experimental import pallas as pl
from jax.experimental.pallas import tpu as pltpu
from jax.experimental.pallas import tpu_sc as plsc

_N_EMB = 8192
_DIM = 256
_BLK = 128
_GW = 128


def _vq_block_kernel(x_ref, xb_ref, cbb_ref, cb_ref,
                     enc_ref, idx_ref, c2_ref):
    @pl.when(pl.program_id(0) == 0)
    def _():
        cb = cb_ref[...]
        c2_ref[...] = jnp.sum(cb * cb, axis=1, keepdims=True)

    x = x_ref[...]
    xbt = xb_ref[...]
    x2 = jnp.sum(x * x, axis=1, keepdims=True)
    x2_row = jnp.transpose(x2)

    mt = jax.lax.dot_general(
        cbb_ref[...], xbt, (((1,), (0,)), ((), ())),
        preferred_element_type=jnp.float32)
    dt = (x2_row - 2.0 * mt) + c2_ref[...]
    lmin = jnp.min(dt, axis=0, keepdims=True)
    iota_c = jax.lax.broadcasted_iota(
        jnp.int32, (_N_EMB, _BLK), 0).astype(jnp.float32)
    cand = jnp.where(dt == lmin, iota_c, jnp.float32(_N_EMB))
    idxt = jnp.min(cand, axis=0, keepdims=True)
    idx_ref[...] = jnp.broadcast_to(idxt, (8, _BLK))
    idx_col = jnp.transpose(idxt)
    iota_l = jax.lax.broadcasted_iota(
        jnp.int32, (_BLK, _N_EMB), 1).astype(jnp.float32)
    enc_ref[...] = (iota_l == idx_col).astype(jnp.float32)


def _sc_gather(codebook, indices, n_rows):
    vector_mesh = plsc.VectorSubcoreMesh(
        core_axis_name="core", subcore_axis_name="subcore")

    @pl.kernel(out_type=jax.ShapeDtypeStruct((n_rows, _DIM), jnp.float32),
               mesh=vector_mesh)
    def gather_kernel(cb_hbm, i_hbm, o_hbm):
        def body(i_vmem, o_vmem):
            pltpu.sync_copy(cb_hbm.at[i_vmem.at[0]], o_vmem)

        pltpu.emit_pipeline(
            body,
            grid=(n_rows // _GW,),
            in_specs=[pl.BlockSpec((1, _GW), index_map=lambda i: (0, i))],
            out_specs=[pl.BlockSpec((_GW, _DIM), index_map=lambda i: (i, 0))],
            core_axis_name="subcore",
            dimension_semantics=(pltpu.PARALLEL,),
        )(i_hbm, o_hbm)

    return gather_kernel(codebook, indices)


def kernel(inputs, codebook):
    input_shape = inputs.shape
    flat = inputs.reshape(-1, _DIM)
    n_rows = flat.shape[0]
    flatbt = flat.astype(jnp.bfloat16).T
    cbb = codebook.astype(jnp.bfloat16)
    enc, idxb = pl.pallas_call(
        _vq_block_kernel,
        grid=(n_rows // _BLK,),
        in_specs=[
            pl.BlockSpec((_BLK, _DIM), lambda i: (i, 0)),
            pl.BlockSpec((_DIM, _BLK), lambda i: (0, i)),
            pl.BlockSpec((_N_EMB, _DIM), lambda i: (0, 0)),
            pl.BlockSpec((_N_EMB, _DIM), lambda i: (0, 0)),
        ],
        out_specs=[
            pl.BlockSpec((_BLK, _N_EMB), lambda i: (i, 0)),
            pl.BlockSpec((8, _BLK), lambda i: (i, 0)),
        ],
        out_shape=[
            jax.ShapeDtypeStruct((n_rows, _N_EMB), jnp.float32),
            jax.ShapeDtypeStruct((8 * (n_rows // _BLK), _BLK), jnp.float32),
        ],
        scratch_shapes=[pltpu.VMEM((_N_EMB, 1), jnp.float32)],
    )(flat, flatbt, cbb, codebook)
    indices = idxb[::8, :].astype(jnp.int32).reshape(1, n_rows)
    q = _sc_gather(codebook, indices, n_rows)
    return q.reshape(input_shape), enc
